# Initial kernel scaffold; baseline (speedup 1.0000x reference)
import jax
import jax.numpy as jnp
from jax.experimental import pallas as pl


def kernel(words, chars, word_table, char_table):
    raise NotImplementedError("write your pallas kernel here")


# speedup vs baseline: 4.2767x; 4.2767x over previous
import functools

import jax
import jax.numpy as jnp
from jax import lax
from jax.experimental import pallas as pl
from jax.experimental.pallas import tpu as pltpu
from jax.experimental.pallas import tpu_sc as plsc

NC = 2
NS = 16
NW = NC * NS

VOCAB_SIZE = 1000000
EMBED_DIM = 64
CHAR_SIZE = 262
CHAR_EMBED_DIM = 16
BATCH = 1024
SEQ = 200
WORD_LEN = 16

N_WORDS = BATCH * SEQ
N_CHARS = BATCH * SEQ * WORD_LEN

W_PER = N_WORDS // NW
C_PER = N_CHARS // NW

IDXV = 128

W_SUB = 5
W_CHUNK = W_SUB * IDXV
W_STEPS = W_PER // W_CHUNK

C_SUB = 16
C_CHUNK = C_SUB * IDXV
C_STEPS = C_PER // C_CHUNK


def _body(words_hbm, chars_hbm, wtab_hbm, ctab_hbm, wout_hbm, cout_hbm,
          widx_v, wrows_v, cidx_v, crows_v, sem):
    wid = lax.axis_index("s") * NC + lax.axis_index("c")

    pltpu.sync_copy(words_hbm.at[wid], widx_v)

    @pl.loop(0, W_STEPS)
    def _word_chunk(c):
        cps = [
            pltpu.async_copy(
                wtab_hbm.at[widx_v.at[c * W_SUB + j]],
                wrows_v.at[pl.ds(j * IDXV, IDXV)],
                sem,
            )
            for j in range(W_SUB)
        ]
        for cp in cps:
            cp.wait()
        pltpu.sync_copy(
            wrows_v, wout_hbm.at[pl.ds(wid * W_PER + c * W_CHUNK, W_CHUNK)]
        )

    @pl.loop(0, C_STEPS)
    def _char_chunk(c):
        pltpu.sync_copy(chars_hbm.at[wid, pl.ds(c * C_SUB, C_SUB)], cidx_v)
        cps = [
            pltpu.async_copy(
                ctab_hbm.at[cidx_v.at[j]],
                crows_v.at[pl.ds(j * IDXV, IDXV)],
                sem,
            )
            for j in range(C_SUB)
        ]
        for cp in cps:
            cp.wait()
        pltpu.sync_copy(
            crows_v, cout_hbm.at[pl.ds(wid * C_PER + c * C_CHUNK, C_CHUNK)]
        )


@jax.jit
def _run(words, chars, word_table, char_table):
    words3 = words.reshape(NW, W_PER // IDXV, IDXV)
    chars3 = chars.reshape(NW, C_PER // IDXV, IDXV)

    f = pl.kernel(
        _body,
        out_type=(
            jax.ShapeDtypeStruct((N_WORDS, EMBED_DIM), jnp.float32),
            jax.ShapeDtypeStruct((N_CHARS, CHAR_EMBED_DIM), jnp.float32),
        ),
        mesh=plsc.VectorSubcoreMesh(core_axis_name="c", subcore_axis_name="s"),
        compiler_params=pltpu.CompilerParams(use_tc_tiling_on_sc=False),
        scratch_types=[
            pltpu.VMEM((W_PER // IDXV, IDXV), jnp.int32),
            pltpu.VMEM((W_CHUNK, EMBED_DIM), jnp.float32),
            pltpu.VMEM((C_SUB, IDXV), jnp.int32),
            pltpu.VMEM((C_CHUNK, CHAR_EMBED_DIM), jnp.float32),
            pltpu.SemaphoreType.DMA,
        ],
    )
    wout, cout = f(words3, chars3, word_table, char_table)
    return (
        wout.reshape(BATCH, SEQ, EMBED_DIM),
        cout.reshape(BATCH, SEQ, WORD_LEN, CHAR_EMBED_DIM),
    )


def kernel(words, chars, word_table, char_table):
    return _run(words, chars, word_table, char_table)


# speedup vs baseline: 4.2860x; 1.0022x over previous
import jax
import jax.numpy as jnp
from jax import lax
from jax.experimental import pallas as pl
from jax.experimental.pallas import tpu as pltpu
from jax.experimental.pallas import tpu_sc as plsc

NC = 2
NS = 16
NW = NC * NS

EMBED_DIM = 64
CHAR_EMBED_DIM = 16
BATCH = 1024
SEQ = 200
WORD_LEN = 16

N_WORDS = BATCH * SEQ
N_CHARS = BATCH * SEQ * WORD_LEN

W_PER = N_WORDS // NW
C_PER = N_CHARS // NW

IDXV = 128

W_SUB = 5
W_STEPS = W_PER // (W_SUB * IDXV)

C_SUB = 8
C_STEPS = C_PER // (C_SUB * IDXV)


def _pipelined_gather(tab_hbm, out_hbm, wid, sub, steps, dim,
                      ibufs, rbufs, gsems, ssems,
                      idx_hbm=None, isem=None, widx=None):
    chunk = sub * IDXV
    obase = wid * steps * chunk

    def idx_rows(c, b):
        if widx is not None:
            return [widx.at[c * sub + j] for j in range(sub)]
        return [ibufs[b].at[j] for j in range(sub)]

    def issue_g(c, b):
        for j, row in enumerate(idx_rows(c, b)):
            pltpu.async_copy(tab_hbm.at[row], rbufs[b].at[pl.ds(j * IDXV, IDXV)],
                             gsems[b])

    def wait_g(b):
        pltpu.make_async_copy(tab_hbm.at[pl.ds(0, chunk)], rbufs[b],
                              gsems[b]).wait()

    def issue_s(c, b):
        pltpu.async_copy(rbufs[b], out_hbm.at[pl.ds(obase + c * chunk, chunk)],
                         ssems[b])

    def wait_s(b):
        pltpu.make_async_copy(rbufs[b], out_hbm.at[pl.ds(0, chunk)],
                              ssems[b]).wait()

    def issue_i(c, b):
        pltpu.async_copy(idx_hbm.at[wid, pl.ds(c * sub, sub)], ibufs[b], isem)

    def wait_i():
        pltpu.make_async_copy(idx_hbm.at[0, pl.ds(0, sub)], ibufs[0],
                              isem).wait()

    if widx is None:
        pltpu.sync_copy(idx_hbm.at[wid, pl.ds(0, sub)], ibufs[0])
    issue_g(0, 0)
    if widx is None:
        issue_i(1, 1)

    @pl.loop(0, steps // 2)
    def _pair(p):
        c0 = p * 2
        wait_g(0)
        issue_s(c0, 0)
        if widx is None:
            wait_i()

        @pl.when(p > 0)
        def _():
            wait_s(1)

        issue_g(c0 + 1, 1)

        if widx is None:
            @pl.when(c0 + 2 < steps)
            def _():
                issue_i(c0 + 2, 0)

        wait_g(1)
        issue_s(c0 + 1, 1)

        @pl.when(c0 + 2 < steps)
        def _():
            if widx is None:
                wait_i()
            wait_s(0)
            issue_g(c0 + 2, 0)

        if widx is None:
            @pl.when(c0 + 3 < steps)
            def _():
                issue_i(c0 + 3, 1)

    wait_s(0)
    wait_s(1)


def _body(words_hbm, chars_hbm, wtab_hbm, ctab_hbm, wout_hbm, cout_hbm,
          widx_v, wrows0, wrows1, cidx0, cidx1, crows0, crows1,
          wgsem0, wgsem1, wssem0, wssem1,
          cgsem0, cgsem1, cssem0, cssem1, cisem):
    wid = lax.axis_index("s") * NC + lax.axis_index("c")

    pltpu.sync_copy(words_hbm.at[wid], widx_v)
    _pipelined_gather(wtab_hbm, wout_hbm, wid, W_SUB, W_STEPS, EMBED_DIM,
                      None, (wrows0, wrows1), (wgsem0, wgsem1),
                      (wssem0, wssem1), widx=widx_v)

    _pipelined_gather(ctab_hbm, cout_hbm, wid, C_SUB, C_STEPS, CHAR_EMBED_DIM,
                      (cidx0, cidx1), (crows0, crows1), (cgsem0, cgsem1),
                      (cssem0, cssem1), idx_hbm=chars_hbm, isem=cisem)


@jax.jit
def _run(words, chars, word_table, char_table):
    words3 = words.reshape(NW, W_PER // IDXV, IDXV)
    chars3 = chars.reshape(NW, C_PER // IDXV, IDXV)

    f = pl.kernel(
        _body,
        out_type=(
            jax.ShapeDtypeStruct((N_WORDS, EMBED_DIM), jnp.float32),
            jax.ShapeDtypeStruct((N_CHARS, CHAR_EMBED_DIM), jnp.float32),
        ),
        mesh=plsc.VectorSubcoreMesh(core_axis_name="c", subcore_axis_name="s"),
        compiler_params=pltpu.CompilerParams(use_tc_tiling_on_sc=False),
        scratch_types=[
            pltpu.VMEM((W_PER // IDXV, IDXV), jnp.int32),
            pltpu.VMEM((W_SUB * IDXV, EMBED_DIM), jnp.float32),
            pltpu.VMEM((W_SUB * IDXV, EMBED_DIM), jnp.float32),
            pltpu.VMEM((C_SUB, IDXV), jnp.int32),
            pltpu.VMEM((C_SUB, IDXV), jnp.int32),
            pltpu.VMEM((C_SUB * IDXV, CHAR_EMBED_DIM), jnp.float32),
            pltpu.VMEM((C_SUB * IDXV, CHAR_EMBED_DIM), jnp.float32),
        ] + [pltpu.SemaphoreType.DMA] * 9,
    )
    wout, cout = f(words3, chars3, word_table, char_table)
    return (
        wout.reshape(BATCH, SEQ, EMBED_DIM),
        cout.reshape(BATCH, SEQ, WORD_LEN, CHAR_EMBED_DIM),
    )


def kernel(words, chars, word_table, char_table):
    return _run(words, chars, word_table, char_table)


# speedup vs baseline: 9.6368x; 2.2484x over previous
import jax
import jax.numpy as jnp
from jax import lax
from jax.experimental import pallas as pl
from jax.experimental.pallas import tpu as pltpu
from jax.experimental.pallas import tpu_sc as plsc

NC = 2
NS = 16
NW = NC * NS

EMBED_DIM = 64
CHAR_EMBED_DIM = 16
CHAR_SIZE = 262
BATCH = 1024
SEQ = 200
WORD_LEN = 16

BB = BATCH // 128
UNITS = SEQ * BB
U_PER = UNITS // NW
PAIRS = U_PER // 2


def _body(widx_hbm, chars_hbm, wtab_hbm, ctab_hbm, wout_hbm, cout_hbm,
          widx_v, wrows0, wrows1, wo0, wo1,
          cidx0, cidx1, ctab_v, co0, co1,
          wgsem, wosem0, wosem1, cisem, cosem0, cosem1):
    wid = lax.axis_index("s") * NC + lax.axis_index("c")
    ubase = wid * U_PER

    iota = lax.iota(jnp.int32, 16)
    rows_g = [iota + g * 16 for g in range(8)]

    pltpu.sync_copy(widx_hbm.at[pl.ds(ubase, U_PER)], widx_v)

    wrows = (wrows0, wrows1)
    wo = (wo0, wo1)
    wosem = (wosem0, wosem1)

    def wgather(t, j):
        pltpu.async_copy(wtab_hbm.at[widx_v.at[t]], wrows[j], wgsem)

    def wgather_wait(j):
        pltpu.make_async_copy(wtab_hbm.at[pl.ds(0, 128)], wrows[j],
                              wgsem).wait()

    def wout_issue(t, j):
        u = ubase + t
        s = u // BB
        bb = u % BB
        pltpu.async_copy(wo[j], wout_hbm.at[s, :, pl.ds(bb, 1)], wosem[j])

    def wout_wait(j):
        pltpu.make_async_copy(wo[j], wout_hbm.at[0, :, pl.ds(0, 1)],
                              wosem[j]).wait()

    def wcompute(j):
        rbuf = wrows[j]
        obuf = wo[j]

        @pl.loop(0, 8)
        def _cb(cb):
            for ci in range(8):
                c = cb * 8 + ci
                cols = jnp.full((16,), 0, jnp.int32) + c
                for g in range(8):
                    vals = plsc.load_gather(rbuf, [rows_g[g], cols])
                    obuf[cb, 0, ci, pl.ds(g * 16, 16)] = vals

    wgather(0, 0)

    @pl.loop(0, PAIRS)
    def _wpair(p):
        for j in range(2):
            t = p * 2 + j
            wgather_wait(j)
            if j == 0:
                wgather(t + 1, 1)
            else:
                @pl.when(p < PAIRS - 1)
                def _():
                    wgather(t + 1, 0)

            @pl.when(p > 0)
            def _():
                wout_wait(j)

            wcompute(j)
            wout_issue(t, j)

    wout_wait(0)
    wout_wait(1)

    pltpu.sync_copy(ctab_hbm, ctab_v)

    cidx = (cidx0, cidx1)
    co = (co0, co1)
    cosem = (cosem0, cosem1)

    def cidx_issue(t, j):
        u = ubase + t
        s = u // BB
        tb = u % BB
        pltpu.async_copy(chars_hbm.at[s, :, pl.ds(tb * 128, 128)], cidx[j],
                         cisem)

    def cidx_wait(j):
        pltpu.make_async_copy(chars_hbm.at[0, :, pl.ds(0, 128)], cidx[j],
                              cisem).wait()

    def cout_issue(t, j):
        u = ubase + t
        s = u // BB
        tb = u % BB
        pltpu.async_copy(co[j], cout_hbm.at[s, :, :, pl.ds(tb, 1)], cosem[j])

    def cout_wait(j):
        pltpu.make_async_copy(co[j], cout_hbm.at[0, :, :, pl.ds(0, 1)],
                              cosem[j]).wait()

    def ccompute(j):
        ibuf = cidx[j]
        obuf = co[j]

        @pl.loop(0, WORD_LEN)
        def _w(w):
            idxv = [ibuf[w, pl.ds(g * 16, 16)] for g in range(8)]
            for tc in range(2):
                for ci in range(8):
                    c = tc * 8 + ci
                    cols = jnp.full((16,), c, jnp.int32)
                    for g in range(8):
                        vals = plsc.load_gather(ctab_v, [cols, idxv[g]])
                        obuf[w, tc, 0, ci, pl.ds(g * 16, 16)] = vals

    u0 = ubase
    pltpu.sync_copy(
        chars_hbm.at[u0 // BB, :, pl.ds((u0 % BB) * 128, 128)], cidx0)
    cidx_issue(1, 1)

    @pl.loop(0, PAIRS)
    def _cpair(p):
        for j in range(2):
            t = p * 2 + j

            if j == 0:
                @pl.when(p > 0)
                def _():
                    cidx_wait(0)

                cidx_issue(t + 1, 1)
            else:
                cidx_wait(1)

                @pl.when(p < PAIRS - 1)
                def _():
                    cidx_issue(t + 1, 0)

            @pl.when(p > 0)
            def _():
                cout_wait(j)

            ccompute(j)
            cout_issue(t, j)

    cout_wait(0)
    cout_wait(1)


@jax.jit
def _run(words, chars, word_table, char_table):
    widx2 = words.T.reshape(UNITS, 128)
    charsT = chars.transpose(1, 2, 0)
    ctabT = char_table.T

    f = pl.kernel(
        _body,
        out_type=(
            jax.ShapeDtypeStruct((SEQ, 8, BB, 8, 128), jnp.float32),
            jax.ShapeDtypeStruct((SEQ, WORD_LEN, 2, BB, 8, 128), jnp.float32),
        ),
        mesh=plsc.VectorSubcoreMesh(core_axis_name="c", subcore_axis_name="s"),
        compiler_params=pltpu.CompilerParams(use_tc_tiling_on_sc=False,
                                             needs_layout_passes=False),
        scratch_types=[
            pltpu.VMEM((U_PER, 128), jnp.int32),
            pltpu.VMEM((128, EMBED_DIM), jnp.float32),
            pltpu.VMEM((128, EMBED_DIM), jnp.float32),
            pltpu.VMEM((8, 1, 8, 128), jnp.float32),
            pltpu.VMEM((8, 1, 8, 128), jnp.float32),
            pltpu.VMEM((WORD_LEN, 128), jnp.int32),
            pltpu.VMEM((WORD_LEN, 128), jnp.int32),
            pltpu.VMEM((CHAR_EMBED_DIM, CHAR_SIZE), jnp.float32),
            pltpu.VMEM((WORD_LEN, 2, 1, 8, 128), jnp.float32),
            pltpu.VMEM((WORD_LEN, 2, 1, 8, 128), jnp.float32),
        ] + [pltpu.SemaphoreType.DMA] * 6,
    )
    kw, kc = f(widx2, charsT, word_table, ctabT)

    word_emb = kw.transpose(2, 4, 0, 1, 3).reshape(BATCH, SEQ, EMBED_DIM)
    char_emb = kc.transpose(3, 5, 0, 1, 2, 4).reshape(
        BATCH, SEQ, WORD_LEN, CHAR_EMBED_DIM)
    return word_emb, char_emb


def kernel(words, chars, word_table, char_table):
    return _run(words, chars, word_table, char_table)
